# baseline (speedup 1.0000x reference)
import functools

import jax
import jax.numpy as jnp
from jax import lax
from jax.experimental import pallas as pl
from jax.experimental.pallas import tpu as pltpu
from jax.experimental.pallas import tpu_sc as plsc

N = 10000
E = 320000
NC = 2
NS = 16
NW = NC * NS
B = 128
ZROWS = 632
ACC_ROWS = ZROWS * NS
R = ACC_ROWS
ZCH = (128, 128, 128, 128, 120)

KLEAD = 4
CH0 = 128
CH1 = 32
E_PAD = NS * (CH0 + CH1) * B
CHD = 80


@functools.lru_cache(maxsize=None)
def _make_agg(d):
  NBUF = 6 if d == 64 else 8

  @functools.partial(
      pl.kernel,
      out_type=jax.ShapeDtypeStruct((NC * R, d), jnp.float32),
      mesh=plsc.VectorSubcoreMesh(core_axis_name="c", subcore_axis_name="s"),
      compiler_params=pltpu.CompilerParams(use_tc_tiling_on_sc=False),
      scratch_types=[
          pltpu.VMEM((CH0, B), jnp.int32),
          pltpu.VMEM((CH0, B), jnp.int32),
          pltpu.VMEM((NBUF, B, d), jnp.float32),
          pltpu.VMEM_SHARED((ACC_ROWS, d), jnp.float32),
          pltpu.SemaphoreType.DMA((NBUF,)),
          pltpu.SemaphoreType.DMA((NBUF,)),
      ],
  )
  def agg(src_hbm, dst_hbm, u_hbm, zeros_hbm, out_hbm, src_v, dst_v, bufs, acc,
          gsem, ssem):
    c = lax.axis_index("c")
    s = lax.axis_index("s")
    off = s * ZROWS

    pltpu.sync_copy(zeros_hbm, bufs.at[0])
    ro = 0
    for rows in ZCH:
      pltpu.sync_copy(bufs.at[0].at[pl.ds(0, rows)],
                      acc.at[pl.ds(off + ro, rows)])
      ro += rows
    plsc.subcore_barrier()

    def run(base, ch):
      pltpu.sync_copy(src_hbm.at[pl.ds(base, ch)], src_v.at[pl.ds(0, ch)])
      pltpu.sync_copy(dst_hbm.at[pl.ds(base, ch)], dst_v.at[pl.ds(0, ch)])

      for j in range(KLEAD):
        pltpu.async_copy(u_hbm.at[src_v.at[j]], bufs.at[j], gsem.at[j])

      def body(i, carry):
        jj = i + KLEAD
        b2 = lax.rem(jj, NBUF)

        @pl.when(jnp.logical_and(jj < ch, jj >= NBUF))
        def _():
          pltpu.make_async_copy(bufs.at[b2], acc.at[dst_v.at[jj - NBUF]],
                                ssem.at[b2]).wait()

        @pl.when(jj < ch)
        def _():
          pltpu.async_copy(u_hbm.at[src_v.at[jj]], bufs.at[b2], gsem.at[b2])

        b = lax.rem(i, NBUF)
        pltpu.make_async_copy(u_hbm.at[src_v.at[i]], bufs.at[b],
                              gsem.at[b]).wait()
        pltpu.async_copy(bufs.at[b], acc.at[dst_v.at[i]], ssem.at[b], add=True)
        return carry

      lax.fori_loop(0, ch, body, 0)
      for j in range(ch - NBUF, ch):
        pltpu.make_async_copy(bufs.at[j % NBUF], acc.at[dst_v.at[j]],
                              ssem.at[j % NBUF]).wait()

    @pl.when(c == 0)
    def _():
      run(s * CH0, CH0)

    @pl.when(c == 1)
    def _():
      run(NS * CH0 + s * CH1, CH1)

    plsc.subcore_barrier()

    ro = 0
    for rows in ZCH:
      pltpu.sync_copy(acc.at[pl.ds(off + ro, rows)],
                      bufs.at[0].at[pl.ds(0, rows)])
      pltpu.sync_copy(bufs.at[0].at[pl.ds(0, rows)],
                      out_hbm.at[pl.ds(c * R + off + ro, rows)])
      ro += rows

  return agg


_DDEG = 8


@functools.lru_cache(maxsize=None)
def _make_deg():
  @functools.partial(
      pl.kernel,
      out_type=jax.ShapeDtypeStruct((NC * R, _DDEG), jnp.float32),
      mesh=plsc.VectorSubcoreMesh(core_axis_name="c", subcore_axis_name="s"),
      compiler_params=pltpu.CompilerParams(use_tc_tiling_on_sc=False),
      scratch_types=[
          pltpu.VMEM((CHD, B), jnp.int32),
          pltpu.VMEM((2 * B, _DDEG), jnp.float32),
          pltpu.VMEM_SHARED((ACC_ROWS, _DDEG), jnp.float32),
      ],
  )
  def _deg_kernel(dst_hbm, oz_hbm, out_hbm, dst_v, buf, acc):
    c = lax.axis_index("c")
    s = lax.axis_index("s")
    w = c * NS + s
    off = s * ZROWS
    pltpu.sync_copy(oz_hbm, buf)
    pltpu.sync_copy(dst_hbm.at[pl.ds(w * CHD, CHD)], dst_v)
    ro = 0
    for rows in ZCH:
      pltpu.sync_copy(buf.at[pl.ds(B, rows)], acc.at[pl.ds(off + ro, rows)])
      ro += rows
    plsc.subcore_barrier()

    def body(j, carry):
      pltpu.sync_copy(buf.at[pl.ds(0, B)], acc.at[dst_v.at[j]], add=True)
      return carry

    lax.fori_loop(0, CHD, body, 0)
    plsc.subcore_barrier()
    ro = 0
    for rows in ZCH:
      pltpu.sync_copy(acc.at[pl.ds(off + ro, rows)], buf.at[pl.ds(B, rows)])
      pltpu.sync_copy(buf.at[pl.ds(B, rows)],
                      out_hbm.at[pl.ds(c * R + off + ro, rows)])
      ro += rows

  return _deg_kernel


def _tc(body, out_shapes):
  return pl.pallas_call(
      body,
      out_shape=[jax.ShapeDtypeStruct(s, jnp.float32) for s in out_shapes])


def _tc1(degp_ref, x_ref, w1_ref, dinv_ref, u1_ref):
  deg = degp_ref[:N, 0] + degp_ref[R:R + N, 0] + 1.0
  dinv = lax.rsqrt(deg)[:, None]
  dinv_ref[...] = dinv
  u1_ref[...] = jnp.dot(x_ref[...], w1_ref[...],
                        preferred_element_type=jnp.float32) * dinv


def _tc2(a1_ref, u1_ref, dinv_ref, b1_ref, w2_ref, u2_ref):
  dinv = dinv_ref[...]
  a1 = a1_ref[0, :N] + a1_ref[1, :N]
  h1 = jnp.maximum(dinv * (a1 + u1_ref[...]) + b1_ref[...], 0.0)
  u2_ref[...] = jnp.dot(h1, w2_ref[...],
                        preferred_element_type=jnp.float32) * dinv


def _tc3(a2_ref, u2_ref, dinv_ref, b2_ref, u3_ref):
  dinv = dinv_ref[...]
  a2 = a2_ref[0, :N] + a2_ref[1, :N]
  z = dinv * (a2 + u2_ref[...]) + b2_ref[...]
  u3_ref[...] = dinv * z


def _tc4(a3_ref, u3_ref, dinv_ref, b3_ref, w3_ref, u4_ref):
  dinv = dinv_ref[...]
  pz = dinv * (a3_ref[0, :N] + a3_ref[1, :N] + u3_ref[...])
  dlay = jnp.maximum(
      jnp.dot(pz, w3_ref[...], preferred_element_type=jnp.float32) +
      b3_ref[...], 0.0)
  u4_ref[...] = dinv * dlay


def _tc5(a4_ref, u4_ref, dinv_ref, b4_ref, w4_ref, xhat_ref):
  dinv = dinv_ref[...]
  pd = dinv * (a4_ref[0, :N] + a4_ref[1, :N] + u4_ref[...])
  xhat_ref[...] = jnp.dot(pd, w4_ref[...],
                          preferred_element_type=jnp.float32) + b4_ref[...]


def kernel(x, edge_index, W_e1, b_e1, W_e2, b_e2, W_d1, b_d1, W_d2, b_d2):
  src = edge_index[0]
  dst = edge_index[1]
  pad = E_PAD - E
  src2d = jnp.concatenate([src, jnp.zeros((pad,), jnp.int32)]).reshape(-1, B)
  dst2d = jnp.concatenate([dst, jnp.full((pad,), N, jnp.int32)]).reshape(-1, B)
  z64 = jnp.zeros((B, 64), jnp.float32)
  z32 = jnp.zeros((B, 32), jnp.float32)
  oz8 = jnp.concatenate([jnp.ones((B, _DDEG), jnp.float32),
                         jnp.zeros((B, _DDEG), jnp.float32)])

  degp = _make_deg()(dst2d, oz8)

  agg64 = _make_agg(64)
  agg32 = _make_agg(32)
  dinv, u1 = _tc(_tc1, [(N, 1), (N, 64)])(degp, x, W_e1)
  a1 = agg64(src2d, dst2d, u1, z64).reshape(NC, R, 64)
  (u2,) = _tc(_tc2, [(N, 32)])(a1, u1, dinv, b_e1[None, :], W_e2)
  a2 = agg32(src2d, dst2d, u2, z32).reshape(NC, R, 32)
  (u3,) = _tc(_tc3, [(N, 32)])(a2, u2, dinv, b_e2[None, :])
  a3 = agg32(src2d, dst2d, u3, z32).reshape(NC, R, 32)
  (u4,) = _tc(_tc4, [(N, 64)])(a3, u3, dinv, b_d1[None, :], W_d1)
  a4 = agg64(src2d, dst2d, u4, z64).reshape(NC, R, 64)
  (xhat,) = _tc(_tc5, [(N, 128)])(a4, u4, dinv, b_d2[None, :], W_d2)
  return xhat


# speedup vs baseline: 1.9729x; 1.9729x over previous
import functools

import jax
import jax.numpy as jnp
from jax import lax
from jax.experimental import pallas as pl
from jax.experimental.pallas import tpu as pltpu
from jax.experimental.pallas import tpu_sc as plsc

N = 10000
E = 320000
NC = 2
NS = 16
NW = NC * NS
B = 128
ZROWS = 632
R = ZROWS * NS
ZCH = (128, 128, 128, 128, 120)

KLEAD = 2
CH = 80
E_PAD = NW * CH * B


@functools.lru_cache(maxsize=None)
def _make_agg(d):
  NBUF = 3 if d == 64 else 6

  @functools.partial(
      pl.kernel,
      out_type=jax.ShapeDtypeStruct((NC * R, d), jnp.float32),
      mesh=plsc.VectorSubcoreMesh(core_axis_name="c", subcore_axis_name="s"),
      compiler_params=pltpu.CompilerParams(use_tc_tiling_on_sc=False),
      scratch_types=[
          pltpu.VMEM((CH, B), jnp.int32),
          pltpu.VMEM((CH, B), jnp.int32),
          pltpu.VMEM((NBUF, B, d), jnp.float32),
          pltpu.VMEM_SHARED((R, d), jnp.float32),
          pltpu.VMEM_SHARED((R, d), jnp.float32),
          pltpu.SemaphoreType.DMA((NBUF,)),
          pltpu.SemaphoreType.DMA((NBUF,)),
      ],
  )
  def agg(src_hbm, dst_hbm, u_hbm, zeros_hbm, out_hbm, src_v, dst_v, bufs,
          u_s, acc, gsem, ssem):
    c = lax.axis_index("c")
    s = lax.axis_index("s")
    w = c * NS + s
    off = s * ZROWS

    pltpu.sync_copy(zeros_hbm, bufs.at[0])
    ro = 0
    for rows in ZCH:
      pltpu.sync_copy(bufs.at[0].at[pl.ds(0, rows)],
                      acc.at[pl.ds(off + ro, rows)])
      pltpu.sync_copy(u_hbm.at[pl.ds(off + ro, rows)],
                      bufs.at[1].at[pl.ds(0, rows)])
      pltpu.sync_copy(bufs.at[1].at[pl.ds(0, rows)],
                      u_s.at[pl.ds(off + ro, rows)])
      ro += rows
    pltpu.sync_copy(src_hbm.at[pl.ds(w * CH, CH)], src_v)
    pltpu.sync_copy(dst_hbm.at[pl.ds(w * CH, CH)], dst_v)
    plsc.subcore_barrier()

    for j in range(KLEAD):
      pltpu.async_copy(u_s.at[src_v.at[j]], bufs.at[j], gsem.at[j])

    def body(i, carry):
      jj = i + KLEAD
      b2 = lax.rem(jj, NBUF)

      @pl.when(jnp.logical_and(jj < CH, jj >= NBUF))
      def _():
        pltpu.make_async_copy(bufs.at[b2], acc.at[dst_v.at[jj - NBUF]],
                              ssem.at[b2]).wait()

      @pl.when(jj < CH)
      def _():
        pltpu.async_copy(u_s.at[src_v.at[jj]], bufs.at[b2], gsem.at[b2])

      b = lax.rem(i, NBUF)
      pltpu.make_async_copy(u_s.at[src_v.at[i]], bufs.at[b],
                            gsem.at[b]).wait()
      pltpu.async_copy(bufs.at[b], acc.at[dst_v.at[i]], ssem.at[b], add=True)
      return carry

    lax.fori_loop(0, CH, body, 0)
    for j in range(CH - NBUF, CH):
      pltpu.make_async_copy(bufs.at[j % NBUF], acc.at[dst_v.at[j]],
                            ssem.at[j % NBUF]).wait()
    plsc.subcore_barrier()

    ro = 0
    for rows in ZCH:
      pltpu.sync_copy(acc.at[pl.ds(off + ro, rows)],
                      bufs.at[0].at[pl.ds(0, rows)])
      pltpu.sync_copy(bufs.at[0].at[pl.ds(0, rows)],
                      out_hbm.at[pl.ds(c * R + off + ro, rows)])
      ro += rows

  return agg


_DDEG = 8


@functools.lru_cache(maxsize=None)
def _make_deg():
  @functools.partial(
      pl.kernel,
      out_type=jax.ShapeDtypeStruct((NC * R, _DDEG), jnp.float32),
      mesh=plsc.VectorSubcoreMesh(core_axis_name="c", subcore_axis_name="s"),
      compiler_params=pltpu.CompilerParams(use_tc_tiling_on_sc=False),
      scratch_types=[
          pltpu.VMEM((CH, B), jnp.int32),
          pltpu.VMEM((2 * B, _DDEG), jnp.float32),
          pltpu.VMEM_SHARED((R, _DDEG), jnp.float32),
      ],
  )
  def _deg_kernel(dst_hbm, oz_hbm, out_hbm, dst_v, buf, acc):
    c = lax.axis_index("c")
    s = lax.axis_index("s")
    w = c * NS + s
    off = s * ZROWS
    pltpu.sync_copy(oz_hbm, buf)
    pltpu.sync_copy(dst_hbm.at[pl.ds(w * CH, CH)], dst_v)
    ro = 0
    for rows in ZCH:
      pltpu.sync_copy(buf.at[pl.ds(B, rows)], acc.at[pl.ds(off + ro, rows)])
      ro += rows
    plsc.subcore_barrier()

    def body(j, carry):
      pltpu.sync_copy(buf.at[pl.ds(0, B)], acc.at[dst_v.at[j]], add=True)
      return carry

    lax.fori_loop(0, CH, body, 0)
    plsc.subcore_barrier()
    ro = 0
    for rows in ZCH:
      pltpu.sync_copy(acc.at[pl.ds(off + ro, rows)], buf.at[pl.ds(B, rows)])
      pltpu.sync_copy(buf.at[pl.ds(B, rows)],
                      out_hbm.at[pl.ds(c * R + off + ro, rows)])
      ro += rows

  return _deg_kernel


def _tc(body, out_shapes):
  return pl.pallas_call(
      body,
      out_shape=[jax.ShapeDtypeStruct(s, jnp.float32) for s in out_shapes])


def _pad_r(v, d):
  return jnp.concatenate([v, jnp.zeros((R - N, d), jnp.float32)], axis=0)


def _tc1(degp_ref, x_ref, w1_ref, dinv_ref, u1_ref):
  deg = degp_ref[:N, 0] + degp_ref[R:R + N, 0] + 1.0
  dinv = lax.rsqrt(deg)[:, None]
  dinv_ref[...] = dinv
  u1 = jnp.dot(x_ref[...], w1_ref[...],
               preferred_element_type=jnp.float32) * dinv
  u1_ref[...] = _pad_r(u1, 64)


def _tc2(a1_ref, u1_ref, dinv_ref, b1_ref, w2_ref, u2_ref):
  dinv = dinv_ref[...]
  a1 = a1_ref[0, :N] + a1_ref[1, :N]
  h1 = jnp.maximum(dinv * (a1 + u1_ref[:N]) + b1_ref[...], 0.0)
  u2 = jnp.dot(h1, w2_ref[...], preferred_element_type=jnp.float32) * dinv
  u2_ref[...] = _pad_r(u2, 32)


def _tc3(a2_ref, u2_ref, dinv_ref, b2_ref, u3_ref):
  dinv = dinv_ref[...]
  a2 = a2_ref[0, :N] + a2_ref[1, :N]
  z = dinv * (a2 + u2_ref[:N]) + b2_ref[...]
  u3_ref[...] = _pad_r(dinv * z, 32)


def _tc4(a3_ref, u3_ref, dinv_ref, b3_ref, w3_ref, u4_ref):
  dinv = dinv_ref[...]
  pz = dinv * (a3_ref[0, :N] + a3_ref[1, :N] + u3_ref[:N])
  dlay = jnp.maximum(
      jnp.dot(pz, w3_ref[...], preferred_element_type=jnp.float32) +
      b3_ref[...], 0.0)
  u4_ref[...] = _pad_r(dinv * dlay, 64)


def _tc5(a4_ref, u4_ref, dinv_ref, b4_ref, w4_ref, xhat_ref):
  dinv = dinv_ref[...]
  pd = dinv * (a4_ref[0, :N] + a4_ref[1, :N] + u4_ref[:N])
  xhat_ref[...] = jnp.dot(pd, w4_ref[...],
                          preferred_element_type=jnp.float32) + b4_ref[...]


def kernel(x, edge_index, W_e1, b_e1, W_e2, b_e2, W_d1, b_d1, W_d2, b_d2):
  src = edge_index[0]
  dst = edge_index[1]
  pad = E_PAD - E
  src2d = jnp.concatenate([src, jnp.zeros((pad,), jnp.int32)]).reshape(-1, B)
  dst2d = jnp.concatenate([dst, jnp.full((pad,), N, jnp.int32)]).reshape(-1, B)
  z64 = jnp.zeros((B, 64), jnp.float32)
  z32 = jnp.zeros((B, 32), jnp.float32)
  oz8 = jnp.concatenate([jnp.ones((B, _DDEG), jnp.float32),
                         jnp.zeros((B, _DDEG), jnp.float32)])

  degp = _make_deg()(dst2d, oz8)

  agg64 = _make_agg(64)
  agg32 = _make_agg(32)
  dinv, u1 = _tc(_tc1, [(N, 1), (R, 64)])(degp, x, W_e1)
  a1 = agg64(src2d, dst2d, u1, z64).reshape(NC, R, 64)
  (u2,) = _tc(_tc2, [(R, 32)])(a1, u1, dinv, b_e1[None, :], W_e2)
  a2 = agg32(src2d, dst2d, u2, z32).reshape(NC, R, 32)
  (u3,) = _tc(_tc3, [(R, 32)])(a2, u2, dinv, b_e2[None, :])
  a3 = agg32(src2d, dst2d, u3, z32).reshape(NC, R, 32)
  (u4,) = _tc(_tc4, [(R, 64)])(a3, u3, dinv, b_d1[None, :], W_d1)
  a4 = agg64(src2d, dst2d, u4, z64).reshape(NC, R, 64)
  (xhat,) = _tc(_tc5, [(N, 128)])(a4, u4, dinv, b_d2[None, :], W_d2)
  return xhat


# speedup vs baseline: 2.0079x; 1.0178x over previous
import functools

import jax
import jax.numpy as jnp
from jax import lax
from jax.experimental import pallas as pl
from jax.experimental.pallas import tpu as pltpu
from jax.experimental.pallas import tpu_sc as plsc

N = 10000
E = 320000
NC = 2
NS = 16
NW = NC * NS
B = 128
ZROWS = 632
R = ZROWS * NS
ZCH = (128, 128, 128, 128, 120)

KLEAD = 2
CH = 80
NCHUNK = E // B
CH_LAST = NCHUNK - 31 * CH


@functools.lru_cache(maxsize=None)
def _make_agg(d):
  NBUF = 3 if d == 64 else 6

  @functools.partial(
      pl.kernel,
      out_type=jax.ShapeDtypeStruct((NC * R, d), jnp.float32),
      mesh=plsc.VectorSubcoreMesh(core_axis_name="c", subcore_axis_name="s"),
      compiler_params=pltpu.CompilerParams(use_tc_tiling_on_sc=False),
      scratch_types=[
          pltpu.VMEM((CH, B), jnp.int32),
          pltpu.VMEM((CH, B), jnp.int32),
          pltpu.VMEM((NBUF, B, d), jnp.float32),
          pltpu.VMEM_SHARED((R, d), jnp.float32),
          pltpu.VMEM_SHARED((R, d), jnp.float32),
          pltpu.SemaphoreType.DMA((NBUF,)),
          pltpu.SemaphoreType.DMA((NBUF,)),
      ],
  )
  def agg(src_hbm, dst_hbm, u_hbm, zeros_hbm, out_hbm, src_v, dst_v, bufs,
          u_s, acc, gsem, ssem):
    c = lax.axis_index("c")
    s = lax.axis_index("s")
    w = c * NS + s
    off = s * ZROWS

    pltpu.sync_copy(zeros_hbm, bufs.at[0])
    ro = 0
    for rows in ZCH:
      pltpu.sync_copy(bufs.at[0].at[pl.ds(0, rows)],
                      acc.at[pl.ds(off + ro, rows)])
      pltpu.sync_copy(u_hbm.at[pl.ds(off + ro, rows)],
                      bufs.at[1].at[pl.ds(0, rows)])
      pltpu.sync_copy(bufs.at[1].at[pl.ds(0, rows)],
                      u_s.at[pl.ds(off + ro, rows)])
      ro += rows
    plsc.subcore_barrier()

    def run(base, ch):
      pltpu.sync_copy(src_hbm.at[pl.ds(base, ch)], src_v.at[pl.ds(0, ch)])
      pltpu.sync_copy(dst_hbm.at[pl.ds(base, ch)], dst_v.at[pl.ds(0, ch)])

      for j in range(KLEAD):
        pltpu.async_copy(u_s.at[src_v.at[j]], bufs.at[j], gsem.at[j])

      def body(i, carry):
        jj = i + KLEAD
        b2 = lax.rem(jj, NBUF)

        @pl.when(jnp.logical_and(jj < ch, jj >= NBUF))
        def _():
          pltpu.make_async_copy(bufs.at[b2], acc.at[dst_v.at[jj - NBUF]],
                                ssem.at[b2]).wait()

        @pl.when(jj < ch)
        def _():
          pltpu.async_copy(u_s.at[src_v.at[jj]], bufs.at[b2], gsem.at[b2])

        b = lax.rem(i, NBUF)
        pltpu.make_async_copy(u_s.at[src_v.at[i]], bufs.at[b],
                              gsem.at[b]).wait()
        pltpu.async_copy(bufs.at[b], acc.at[dst_v.at[i]], ssem.at[b], add=True)
        return carry

      lax.fori_loop(0, ch, body, 0)
      for j in range(ch - NBUF, ch):
        pltpu.make_async_copy(bufs.at[j % NBUF], acc.at[dst_v.at[j]],
                              ssem.at[j % NBUF]).wait()

    @pl.when(w < 31)
    def _():
      run(w * CH, CH)

    @pl.when(w == 31)
    def _():
      run(31 * CH, CH_LAST)

    plsc.subcore_barrier()

    ro = 0
    for rows in ZCH:
      pltpu.sync_copy(acc.at[pl.ds(off + ro, rows)],
                      bufs.at[0].at[pl.ds(0, rows)])
      pltpu.sync_copy(bufs.at[0].at[pl.ds(0, rows)],
                      out_hbm.at[pl.ds(c * R + off + ro, rows)])
      ro += rows

  return agg


_DDEG = 8


@functools.lru_cache(maxsize=None)
def _make_deg():
  @functools.partial(
      pl.kernel,
      out_type=jax.ShapeDtypeStruct((NC * R, _DDEG), jnp.float32),
      mesh=plsc.VectorSubcoreMesh(core_axis_name="c", subcore_axis_name="s"),
      compiler_params=pltpu.CompilerParams(use_tc_tiling_on_sc=False),
      scratch_types=[
          pltpu.VMEM((CH, B), jnp.int32),
          pltpu.VMEM((2 * B, _DDEG), jnp.float32),
          pltpu.VMEM_SHARED((R, _DDEG), jnp.float32),
      ],
  )
  def _deg_kernel(dst_hbm, oz_hbm, out_hbm, dst_v, buf, acc):
    c = lax.axis_index("c")
    s = lax.axis_index("s")
    w = c * NS + s
    off = s * ZROWS
    pltpu.sync_copy(oz_hbm, buf)
    ro = 0
    for rows in ZCH:
      pltpu.sync_copy(buf.at[pl.ds(B, rows)], acc.at[pl.ds(off + ro, rows)])
      ro += rows
    plsc.subcore_barrier()

    def run(base, ch):
      pltpu.sync_copy(dst_hbm.at[pl.ds(base, ch)], dst_v.at[pl.ds(0, ch)])

      def body(j, carry):
        pltpu.sync_copy(buf.at[pl.ds(0, B)], acc.at[dst_v.at[j]], add=True)
        return carry

      lax.fori_loop(0, ch, body, 0)

    @pl.when(w < 31)
    def _():
      run(w * CH, CH)

    @pl.when(w == 31)
    def _():
      run(31 * CH, CH_LAST)

    plsc.subcore_barrier()
    ro = 0
    for rows in ZCH:
      pltpu.sync_copy(acc.at[pl.ds(off + ro, rows)], buf.at[pl.ds(B, rows)])
      pltpu.sync_copy(buf.at[pl.ds(B, rows)],
                      out_hbm.at[pl.ds(c * R + off + ro, rows)])
      ro += rows

  return _deg_kernel


def _tc(body, out_shapes):
  return pl.pallas_call(
      body,
      out_shape=[jax.ShapeDtypeStruct(s, jnp.float32) for s in out_shapes])


def _pad_r(v, d):
  return jnp.concatenate([v, jnp.zeros((R - N, d), jnp.float32)], axis=0)


def _tc0(x_ref, w1_ref, mm_ref):
  mm_ref[...] = jnp.dot(x_ref[...], w1_ref[...],
                        preferred_element_type=jnp.float32)


def _tc1(degp_ref, mm_ref, dinv_ref, u1_ref):
  deg = degp_ref[:N, 0] + degp_ref[R:R + N, 0] + 1.0
  dinv = lax.rsqrt(deg)[:, None]
  dinv_ref[...] = dinv
  u1_ref[...] = _pad_r(mm_ref[...] * dinv, 64)


def _tc2(a1_ref, u1_ref, dinv_ref, b1_ref, w2_ref, u2_ref):
  dinv = dinv_ref[...]
  a1 = a1_ref[0, :N] + a1_ref[1, :N]
  h1 = jnp.maximum(dinv * (a1 + u1_ref[:N]) + b1_ref[...], 0.0)
  u2 = jnp.dot(h1, w2_ref[...], preferred_element_type=jnp.float32) * dinv
  u2_ref[...] = _pad_r(u2, 32)


def _tc3(a2_ref, u2_ref, dinv_ref, b2_ref, u3_ref):
  dinv = dinv_ref[...]
  a2 = a2_ref[0, :N] + a2_ref[1, :N]
  z = dinv * (a2 + u2_ref[:N]) + b2_ref[...]
  u3_ref[...] = _pad_r(dinv * z, 32)


def _tc4(a3_ref, u3_ref, dinv_ref, b3_ref, w3_ref, u4_ref):
  dinv = dinv_ref[...]
  pz = dinv * (a3_ref[0, :N] + a3_ref[1, :N] + u3_ref[:N])
  dlay = jnp.maximum(
      jnp.dot(pz, w3_ref[...], preferred_element_type=jnp.float32) +
      b3_ref[...], 0.0)
  u4_ref[...] = _pad_r(dinv * dlay, 64)


def _tc5(a4_ref, u4_ref, dinv_ref, b4_ref, w4_ref, xhat_ref):
  dinv = dinv_ref[...]
  pd = dinv * (a4_ref[0, :N] + a4_ref[1, :N] + u4_ref[:N])
  xhat_ref[...] = jnp.dot(pd, w4_ref[...],
                          preferred_element_type=jnp.float32) + b4_ref[...]


def kernel(x, edge_index, W_e1, b_e1, W_e2, b_e2, W_d1, b_d1, W_d2, b_d2):
  src2d = edge_index[0].reshape(-1, B)
  dst2d = edge_index[1].reshape(-1, B)
  z64 = jnp.zeros((B, 64), jnp.float32)
  z32 = jnp.zeros((B, 32), jnp.float32)
  oz8 = jnp.concatenate([jnp.ones((B, _DDEG), jnp.float32),
                         jnp.zeros((B, _DDEG), jnp.float32)])

  (mm1,) = _tc(_tc0, [(N, 64)])(x, W_e1)
  degp = _make_deg()(dst2d, oz8)

  agg64 = _make_agg(64)
  agg32 = _make_agg(32)
  dinv, u1 = _tc(_tc1, [(N, 1), (R, 64)])(degp, mm1)
  a1 = agg64(src2d, dst2d, u1, z64).reshape(NC, R, 64)
  (u2,) = _tc(_tc2, [(R, 32)])(a1, u1, dinv, b_e1[None, :], W_e2)
  a2 = agg32(src2d, dst2d, u2, z32).reshape(NC, R, 32)
  (u3,) = _tc(_tc3, [(R, 32)])(a2, u2, dinv, b_e2[None, :])
  a3 = agg32(src2d, dst2d, u3, z32).reshape(NC, R, 32)
  (u4,) = _tc(_tc4, [(R, 64)])(a3, u3, dinv, b_d1[None, :], W_d1)
  a4 = agg64(src2d, dst2d, u4, z64).reshape(NC, R, 64)
  (xhat,) = _tc(_tc5, [(N, 128)])(a4, u4, dinv, b_d2[None, :], W_d2)
  return xhat


# speedup vs baseline: 2.0081x; 1.0001x over previous
import functools

import jax
import jax.numpy as jnp
from jax import lax
from jax.experimental import pallas as pl
from jax.experimental.pallas import tpu as pltpu
from jax.experimental.pallas import tpu_sc as plsc

N = 10000
E = 320000
NC = 2
NS = 16
B = 128
ZROWS = 632
R = ZROWS * NS
ZCH = (128, 128, 128, 128, 120)

KLEAD = 2
CH = 80
NCHUNK = E // B
CH_LAST = NCHUNK - 31 * CH


@functools.lru_cache(maxsize=None)
def _make_agg(d):
  NBUF = 3 if d == 64 else 6

  @functools.partial(
      pl.kernel,
      out_type=jax.ShapeDtypeStruct((NC * R, d), jnp.float32),
      mesh=plsc.VectorSubcoreMesh(core_axis_name="c", subcore_axis_name="s"),
      compiler_params=pltpu.CompilerParams(use_tc_tiling_on_sc=False),
      scratch_types=[
          pltpu.VMEM((CH, B), jnp.int32),
          pltpu.VMEM((CH, B), jnp.int32),
          pltpu.VMEM((NBUF, B, d), jnp.float32),
          pltpu.VMEM_SHARED((R, d), jnp.float32),
          pltpu.VMEM_SHARED((R, d), jnp.float32),
          pltpu.SemaphoreType.DMA((NBUF,)),
          pltpu.SemaphoreType.DMA((NBUF,)),
      ],
  )
  def agg(src_hbm, dst_hbm, u_hbm, zeros_hbm, out_hbm, src_v, dst_v, bufs,
          u_s, acc, gsem, ssem):
    c = lax.axis_index("c")
    s = lax.axis_index("s")
    w = c * NS + s
    off = s * ZROWS

    pltpu.sync_copy(zeros_hbm, bufs.at[0])
    ro = 0
    for rows in ZCH:
      pltpu.sync_copy(bufs.at[0].at[pl.ds(0, rows)],
                      acc.at[pl.ds(off + ro, rows)])
      pltpu.sync_copy(u_hbm.at[pl.ds(off + ro, rows)],
                      bufs.at[1].at[pl.ds(0, rows)])
      pltpu.sync_copy(bufs.at[1].at[pl.ds(0, rows)],
                      u_s.at[pl.ds(off + ro, rows)])
      ro += rows
    plsc.subcore_barrier()

    def run(base, ch):
      pltpu.sync_copy(src_hbm.at[pl.ds(base, ch)], src_v.at[pl.ds(0, ch)])
      pltpu.sync_copy(dst_hbm.at[pl.ds(base, ch)], dst_v.at[pl.ds(0, ch)])

      for j in range(KLEAD):
        pltpu.async_copy(u_s.at[src_v.at[j]], bufs.at[j], gsem.at[j])

      def body(i, carry):
        jj = i + KLEAD
        b2 = lax.rem(jj, NBUF)

        @pl.when(jnp.logical_and(jj < ch, jj >= NBUF))
        def _():
          pltpu.make_async_copy(bufs.at[b2], acc.at[dst_v.at[jj - NBUF]],
                                ssem.at[b2]).wait()

        @pl.when(jj < ch)
        def _():
          pltpu.async_copy(u_s.at[src_v.at[jj]], bufs.at[b2], gsem.at[b2])

        b = lax.rem(i, NBUF)
        pltpu.make_async_copy(u_s.at[src_v.at[i]], bufs.at[b],
                              gsem.at[b]).wait()
        pltpu.async_copy(bufs.at[b], acc.at[dst_v.at[i]], ssem.at[b], add=True)
        return carry

      lax.fori_loop(0, ch, body, 0)
      for j in range(ch - NBUF, ch):
        pltpu.make_async_copy(bufs.at[j % NBUF], acc.at[dst_v.at[j]],
                              ssem.at[j % NBUF]).wait()

    @pl.when(w < 31)
    def _():
      run(w * CH, CH)

    @pl.when(w == 31)
    def _():
      run(31 * CH, CH_LAST)

    plsc.subcore_barrier()

    ro = 0
    for rows in ZCH:
      pltpu.sync_copy(acc.at[pl.ds(off + ro, rows)],
                      bufs.at[0].at[pl.ds(0, rows)])
      pltpu.sync_copy(bufs.at[0].at[pl.ds(0, rows)],
                      out_hbm.at[pl.ds(c * R + off + ro, rows)])
      ro += rows

  return agg


_DDEG = 8


@functools.lru_cache(maxsize=None)
def _make_deg():
  @functools.partial(
      pl.kernel,
      out_type=jax.ShapeDtypeStruct((NC * R, _DDEG), jnp.float32),
      mesh=plsc.VectorSubcoreMesh(core_axis_name="c", subcore_axis_name="s"),
      compiler_params=pltpu.CompilerParams(use_tc_tiling_on_sc=False),
      scratch_types=[
          pltpu.VMEM((CH, B), jnp.int32),
          pltpu.VMEM((2 * B, _DDEG), jnp.float32),
          pltpu.VMEM_SHARED((R, _DDEG), jnp.float32),
      ],
  )
  def _deg_kernel(dst_hbm, oz_hbm, out_hbm, dst_v, buf, acc):
    c = lax.axis_index("c")
    s = lax.axis_index("s")
    w = c * NS + s
    off = s * ZROWS
    pltpu.sync_copy(oz_hbm, buf)
    ro = 0
    for rows in ZCH:
      pltpu.sync_copy(buf.at[pl.ds(B, rows)], acc.at[pl.ds(off + ro, rows)])
      ro += rows
    plsc.subcore_barrier()

    def run(base, ch):
      pltpu.sync_copy(dst_hbm.at[pl.ds(base, ch)], dst_v.at[pl.ds(0, ch)])

      def body(j, carry):
        pltpu.sync_copy(buf.at[pl.ds(0, B)], acc.at[dst_v.at[j]], add=True)
        return carry

      lax.fori_loop(0, ch, body, 0)

    @pl.when(w < 31)
    def _():
      run(w * CH, CH)

    @pl.when(w == 31)
    def _():
      run(31 * CH, CH_LAST)

    plsc.subcore_barrier()
    ro = 0
    for rows in ZCH:
      pltpu.sync_copy(acc.at[pl.ds(off + ro, rows)], buf.at[pl.ds(B, rows)])
      pltpu.sync_copy(buf.at[pl.ds(B, rows)],
                      out_hbm.at[pl.ds(c * R + off + ro, rows)])
      ro += rows

  return _deg_kernel


def _tc(body, out_shapes):
  return pl.pallas_call(
      body,
      out_shape=[jax.ShapeDtypeStruct(s, jnp.float32) for s in out_shapes])


def _pad_r(v, d):
  return jnp.concatenate([v, jnp.zeros((R - N, d), jnp.float32)], axis=0)


def _tc0(x_ref, w1_ref, mm_ref):
  mm_ref[...] = jnp.dot(x_ref[...], w1_ref[...],
                        preferred_element_type=jnp.float32)


def _tc1(degp_ref, mm_ref, dinv_ref, u1_ref):
  deg = degp_ref[:N, 0] + degp_ref[R:R + N, 0] + 1.0
  dinv = lax.rsqrt(deg)[:, None]
  dinv_ref[...] = dinv
  u1_ref[...] = _pad_r(mm_ref[...] * dinv, 64)


def _tc2(a1_ref, u1_ref, dinv_ref, b1_ref, w2_ref, u2_ref):
  dinv = dinv_ref[...]
  a1 = a1_ref[0, :N] + a1_ref[1, :N]
  h1 = jnp.maximum(dinv * (a1 + u1_ref[:N]) + b1_ref[...], 0.0)
  u2 = jnp.dot(h1, w2_ref[...], preferred_element_type=jnp.float32) * dinv
  u2_ref[...] = _pad_r(u2, 32)


def _tc3(a2_ref, u2_ref, dinv_ref, b2_ref, u3_ref):
  dinv = dinv_ref[...]
  a2 = a2_ref[0, :N] + a2_ref[1, :N]
  z = dinv * (a2 + u2_ref[:N]) + b2_ref[...]
  u3_ref[...] = _pad_r(dinv * z, 32)


def _tc4(a3_ref, u3_ref, dinv_ref, b3_ref, w3_ref, u4_ref):
  dinv = dinv_ref[...]
  pz = dinv * (a3_ref[0, :N] + a3_ref[1, :N] + u3_ref[:N])
  dlay = jnp.maximum(
      jnp.dot(pz, w3_ref[...], preferred_element_type=jnp.float32) +
      b3_ref[...], 0.0)
  u4_ref[...] = _pad_r(dinv * dlay, 64)


def _tc5(a4_ref, u4_ref, dinv_ref, b4_ref, w4_ref, xhat_ref):
  dinv = dinv_ref[...]
  pd = dinv * (a4_ref[0, :N] + a4_ref[1, :N] + u4_ref[:N])
  xhat_ref[...] = jnp.dot(pd, w4_ref[...],
                          preferred_element_type=jnp.float32) + b4_ref[...]


def kernel(x, edge_index, W_e1, b_e1, W_e2, b_e2, W_d1, b_d1, W_d2, b_d2):
  src2d = edge_index[0].reshape(-1, B)
  dst2d = edge_index[1].reshape(-1, B)
  z64 = jnp.zeros((B, 64), jnp.float32)
  z32 = jnp.zeros((B, 32), jnp.float32)
  oz8 = jnp.concatenate([jnp.ones((B, _DDEG), jnp.float32),
                         jnp.zeros((B, _DDEG), jnp.float32)])

  (mm1,) = _tc(_tc0, [(N, 64)])(x, W_e1)
  degp = _make_deg()(dst2d, oz8)

  agg64 = _make_agg(64)
  agg32 = _make_agg(32)
  dinv, u1 = _tc(_tc1, [(N, 1), (R, 64)])(degp, mm1)
  a1 = agg64(src2d, dst2d, u1, z64).reshape(NC, R, 64)
  (u2,) = _tc(_tc2, [(R, 32)])(a1, u1, dinv, b_e1[None, :], W_e2)
  a2 = agg32(src2d, dst2d, u2, z32).reshape(NC, R, 32)
  (u3,) = _tc(_tc3, [(R, 32)])(a2, u2, dinv, b_e2[None, :])
  a3 = agg32(src2d, dst2d, u3, z32).reshape(NC, R, 32)
  (u4,) = _tc(_tc4, [(R, 64)])(a3, u3, dinv, b_d1[None, :], W_d1)
  a4 = agg64(src2d, dst2d, u4, z64).reshape(NC, R, 64)
  (xhat,) = _tc(_tc5, [(N, 128)])(a4, u4, dinv, b_d2[None, :], W_d2)
  return xhat
